# baseline (speedup 1.0000x reference)
import functools

import jax
import jax.numpy as jnp
from jax import lax
from jax.experimental import pallas as pl
from jax.experimental.pallas import tpu as pltpu

_K = 100
_PEAK_EPSILON = 1e-06


def _detgen_kernel(heat_ref, size_ref, off_ref,
                   sc_out_ref, box_out_ref, int_out_ref,
                   peaks_ref, cand_ref, fid_ref, pix_smem, *, H, W, C, K):
    HW = H * W
    x = heat_ref[0]
    p = jax.nn.sigmoid(x)

    neg_w = jnp.full((H, 1, C), -jnp.inf, dtype=jnp.float32)
    left = jnp.concatenate([neg_w, p[:, :-1, :]], axis=1)
    right = jnp.concatenate([p[:, 1:, :], neg_w], axis=1)
    mw = jnp.maximum(p, jnp.maximum(left, right))
    neg_h = jnp.full((1, W, C), -jnp.inf, dtype=jnp.float32)
    up = jnp.concatenate([neg_h, mw[:-1]], axis=0)
    dn = jnp.concatenate([mw[1:], neg_h], axis=0)
    m = jnp.maximum(mw, jnp.maximum(up, dn))

    peaks = jnp.where(jnp.abs(p - m) < _PEAK_EPSILON, p, 0.0)
    peaks_ref[...] = peaks.reshape(HW // 8, 8, C)

    colmax = jnp.max(peaks, axis=2)

    pix_iota = (lax.broadcasted_iota(jnp.int32, (H, W), 0) * W
                + lax.broadcasted_iota(jnp.int32, (H, W), 1))
    k_iota = lax.broadcasted_iota(jnp.int32, (1, H), 1)
    s_iota = lax.broadcasted_iota(jnp.int32, (1, 8, C), 1)
    c_iota = lax.broadcasted_iota(jnp.int32, (1, 8, C), 2)
    BIG = jnp.int32(1 << 30)

    def pixel_body(i, cmax):
        gmax = jnp.max(cmax)
        pix = jnp.min(jnp.where(cmax == gmax, pix_iota, BIG))
        pix_smem[i] = pix
        return jnp.where(pix_iota == pix, -1.0, cmax)

    lax.fori_loop(0, K, pixel_body, colmax)

    cand_ref[...] = jnp.full(cand_ref.shape, -1.0, dtype=jnp.float32)
    fid_ref[...] = jnp.full(fid_ref.shape, BIG, dtype=jnp.int32)

    def gather_cand_body(j, _):
        pixj = pix_smem[j]
        g = pixj // 8
        s = pixj - g * 8
        jg = j // 8
        js = j - jg * 8
        page = peaks_ref[pl.ds(g, 1)]
        row = jnp.max(jnp.where(s_iota == s, page, -jnp.inf), axis=1,
                      keepdims=True)
        insl = s_iota == js
        cpage = cand_ref[pl.ds(jg, 1)]
        cand_ref[pl.ds(jg, 1)] = jnp.where(insl, row, cpage)
        fpage = fid_ref[pl.ds(jg, 1)]
        fid_ref[pl.ds(jg, 1)] = jnp.where(insl, pixj * C + c_iota, fpage)
        return 0

    lax.fori_loop(0, K, gather_cand_body, 0)

    cand = cand_ref[...]
    cfid = fid_ref[...]

    def elem_body(i, state):
        vals, sc_acc, id_acc = state
        gmax = jnp.max(vals)
        fpos = jnp.min(jnp.where(vals == gmax, cfid, BIG))
        vals = jnp.where(cfid == fpos, -1.0, vals)
        sc_acc = jnp.where(k_iota == i, gmax, sc_acc)
        id_acc = jnp.where(k_iota == i, fpos, id_acc)
        return vals, sc_acc, id_acc

    sc0 = jnp.zeros((1, H), dtype=jnp.float32)
    id0 = jnp.zeros((1, H), dtype=jnp.int32)
    _, sc, fid = lax.fori_loop(0, K, elem_body, (cand, sc0, id0))

    q = fid // C
    yv = q // W
    xv = q - yv * W
    cv = fid - q * C

    qT = q.reshape(H, 1)
    yT = qT // W
    xT = qT - yT * W
    lane_h = lax.broadcasted_iota(jnp.int32, (H, H), 1)
    onehot = (yT == lane_h).astype(jnp.float32)
    size_rows = jnp.dot(onehot, size_ref[0],
                        preferred_element_type=jnp.float32)
    off_rows = jnp.dot(onehot, off_ref[0],
                       preferred_element_type=jnp.float32)
    lane2 = lax.broadcasted_iota(jnp.int32, (H, 2 * W), 1)
    sel_h = lane2 == 2 * xT
    sel_w = lane2 == 2 * xT + 1
    zf = jnp.float32(0)
    h = jnp.sum(jnp.where(sel_h, size_rows, zf), axis=1, keepdims=True)
    w = jnp.sum(jnp.where(sel_w, size_rows, zf), axis=1, keepdims=True)
    yo = jnp.sum(jnp.where(sel_h, off_rows, zf), axis=1, keepdims=True)
    xo = jnp.sum(jnp.where(sel_w, off_rows, zf), axis=1, keepdims=True)

    yf = yT.astype(jnp.float32)
    xf = xT.astype(jnp.float32)
    hh = jnp.maximum(h, 0.0)
    ww = jnp.maximum(w, 0.0)
    Hf = jnp.float32(H)
    Wf = jnp.float32(W)
    ymin = jnp.clip(yf + yo - hh / 2.0, 0.0, Hf)
    xmin = jnp.clip(xf + xo - ww / 2.0, 0.0, Wf)
    ymax = jnp.clip(yf + yo + hh / 2.0, 0.0, Hf)
    xmax = jnp.clip(xf + xo + ww / 2.0, 0.0, Wf)
    box = jnp.concatenate([ymin, xmin, ymax, xmax], axis=1)
    box = jnp.clip(box * 4.0 / 512.0, 0.0, 1.0)

    nd = jnp.sum(jnp.where((sc > 0.0) & (k_iota < K), 1, 0))
    nd_row = jnp.where(k_iota == 0, nd, 0)

    sc_out_ref[0, 0] = sc[0]
    box_out_ref[0] = box
    int_out_ref[0] = jnp.concatenate([cv, nd_row], axis=0)


def kernel(ct_heatmaps, ct_size, ct_offset):
    B, H, W, C = ct_heatmaps.shape
    K = _K
    size_r = ct_size.reshape(B, H, 2 * W)
    off_r = ct_offset.reshape(B, H, 2 * W)

    body = functools.partial(_detgen_kernel, H=H, W=W, C=C, K=K)
    sc, box, ints = pl.pallas_call(
        body,
        grid=(B,),
        in_specs=[
            pl.BlockSpec((1, H, W, C), lambda b: (b, 0, 0, 0)),
            pl.BlockSpec((1, H, 2 * W), lambda b: (b, 0, 0)),
            pl.BlockSpec((1, H, 2 * W), lambda b: (b, 0, 0)),
        ],
        out_specs=[
            pl.BlockSpec((1, 1, H), lambda b: (b, 0, 0)),
            pl.BlockSpec((1, H, 4), lambda b: (b, 0, 0)),
            pl.BlockSpec((1, 2, H), lambda b: (b, 0, 0)),
        ],
        out_shape=[
            jax.ShapeDtypeStruct((B, 1, H), jnp.float32),
            jax.ShapeDtypeStruct((B, H, 4), jnp.float32),
            jax.ShapeDtypeStruct((B, 2, H), jnp.int32),
        ],
        scratch_shapes=[
            pltpu.VMEM((H * W // 8, 8, C), jnp.float32),
            pltpu.VMEM(((K + 7) // 8, 8, C), jnp.float32),
            pltpu.VMEM(((K + 7) // 8, 8, C), jnp.int32),
            pltpu.SMEM((K,), jnp.int32),
        ],
        compiler_params=pltpu.CompilerParams(
            dimension_semantics=("parallel",),
        ),
    )(ct_heatmaps, size_r, off_r)

    boxes = box[:, :K, :]
    channel_indices = ints[:, 0, :K]
    detection_scores = sc[:, 0, :K]
    num_detections = ints[:, 1, 0]
    return boxes, channel_indices, detection_scores, num_detections


# speedup vs baseline: 1.0190x; 1.0190x over previous
import functools

import jax
import jax.numpy as jnp
from jax import lax
from jax.experimental import pallas as pl
from jax.experimental.pallas import tpu as pltpu

_K = 100
_PEAK_EPSILON = 1e-06


def _detgen_kernel(heat_ref, size_ref, off_ref,
                   sc_out_ref, box_out_ref, int_out_ref,
                   peaks_ref, cand_ref, fid_ref, pix_smem, *, H, W, C, K):
    HW = H * W
    x = heat_ref[0]
    p = jax.nn.sigmoid(x)

    neg_w = jnp.full((H, 1, C), -jnp.inf, dtype=jnp.float32)
    left = jnp.concatenate([neg_w, p[:, :-1, :]], axis=1)
    right = jnp.concatenate([p[:, 1:, :], neg_w], axis=1)
    mw = jnp.maximum(p, jnp.maximum(left, right))
    neg_h = jnp.full((1, W, C), -jnp.inf, dtype=jnp.float32)
    up = jnp.concatenate([neg_h, mw[:-1]], axis=0)
    dn = jnp.concatenate([mw[1:], neg_h], axis=0)
    m = jnp.maximum(mw, jnp.maximum(up, dn))

    peaks = jnp.where(jnp.abs(p - m) < _PEAK_EPSILON, p, 0.0)
    peaks_ref[...] = peaks.reshape(HW // 8, 8, C)

    colmax = jnp.max(peaks, axis=2)

    pix_iota = (lax.broadcasted_iota(jnp.int32, (H, W), 0) * W
                + lax.broadcasted_iota(jnp.int32, (H, W), 1))
    k_iota = lax.broadcasted_iota(jnp.int32, (1, H), 1)
    s_iota = lax.broadcasted_iota(jnp.int32, (1, 8, C), 1)
    c_iota = lax.broadcasted_iota(jnp.int32, (1, 8, C), 2)
    BIG = jnp.int32(1 << 30)

    def pixel_body(i, cmax):
        gmaxv = jnp.max(cmax, axis=(0, 1), keepdims=True)
        pixv = jnp.min(jnp.where(cmax == gmaxv, pix_iota, BIG),
                       axis=(0, 1), keepdims=True)
        pix_smem[i] = pixv[0, 0]
        return jnp.where(pix_iota == pixv, -1.0, cmax)

    lax.fori_loop(0, K, pixel_body, colmax, unroll=4)

    KP = (K + 7) // 8
    c_iota1 = lax.broadcasted_iota(jnp.int32, (1, 1, C), 2)

    def gather_cand_body(j, _):
        base = j * 8
        rows, fids = [], []
        for t in range(8):
            pixj = pix_smem[jnp.minimum(base + t, K - 1)]
            g = pixj // 8
            s = pixj - g * 8
            page = peaks_ref[pl.ds(g, 1)]
            rows.append(jnp.max(jnp.where(s_iota == s, page, -jnp.inf),
                                axis=1, keepdims=True))
            fids.append(pixj * C + c_iota1)
        valid = s_iota < K - base
        cand_ref[pl.ds(j, 1)] = jnp.where(
            valid, jnp.concatenate(rows, axis=1), -1.0)
        fid_ref[pl.ds(j, 1)] = jnp.where(
            valid, jnp.concatenate(fids, axis=1), BIG)
        return 0

    lax.fori_loop(0, KP, gather_cand_body, 0)

    cand = cand_ref[...]
    cfid = fid_ref[...]

    def elem_body(i, state):
        vals, sc_acc, id_acc = state
        gmaxv = jnp.max(vals, axis=(0, 1, 2), keepdims=True)
        fposv = jnp.min(jnp.where(vals == gmaxv, cfid, BIG),
                        axis=(0, 1, 2), keepdims=True)
        vals = jnp.where(cfid == fposv, -1.0, vals)
        sc_acc = jnp.where(k_iota == i, gmaxv[0], sc_acc)
        id_acc = jnp.where(k_iota == i, fposv[0], id_acc)
        return vals, sc_acc, id_acc

    sc0 = jnp.zeros((1, H), dtype=jnp.float32)
    id0 = jnp.zeros((1, H), dtype=jnp.int32)
    _, sc, fid = lax.fori_loop(0, K, elem_body, (cand, sc0, id0), unroll=4)

    q = fid // C
    yv = q // W
    xv = q - yv * W
    cv = fid - q * C

    qT = q.reshape(H, 1)
    yT = qT // W
    xT = qT - yT * W
    lane_h = lax.broadcasted_iota(jnp.int32, (H, H), 1)
    onehot = (yT == lane_h).astype(jnp.float32)
    size_rows = jnp.dot(onehot, size_ref[0],
                        preferred_element_type=jnp.float32)
    off_rows = jnp.dot(onehot, off_ref[0],
                       preferred_element_type=jnp.float32)
    lane2 = lax.broadcasted_iota(jnp.int32, (H, 2 * W), 1)
    sel_h = lane2 == 2 * xT
    sel_w = lane2 == 2 * xT + 1
    zf = jnp.float32(0)
    h = jnp.sum(jnp.where(sel_h, size_rows, zf), axis=1, keepdims=True)
    w = jnp.sum(jnp.where(sel_w, size_rows, zf), axis=1, keepdims=True)
    yo = jnp.sum(jnp.where(sel_h, off_rows, zf), axis=1, keepdims=True)
    xo = jnp.sum(jnp.where(sel_w, off_rows, zf), axis=1, keepdims=True)

    yf = yT.astype(jnp.float32)
    xf = xT.astype(jnp.float32)
    hh = jnp.maximum(h, 0.0)
    ww = jnp.maximum(w, 0.0)
    Hf = jnp.float32(H)
    Wf = jnp.float32(W)
    ymin = jnp.clip(yf + yo - hh / 2.0, 0.0, Hf)
    xmin = jnp.clip(xf + xo - ww / 2.0, 0.0, Wf)
    ymax = jnp.clip(yf + yo + hh / 2.0, 0.0, Hf)
    xmax = jnp.clip(xf + xo + ww / 2.0, 0.0, Wf)
    box = jnp.concatenate([ymin, xmin, ymax, xmax], axis=1)
    box = jnp.clip(box * 4.0 / 512.0, 0.0, 1.0)

    nd = jnp.sum(jnp.where((sc > 0.0) & (k_iota < K), 1, 0))
    nd_row = jnp.where(k_iota == 0, nd, 0)

    sc_out_ref[0, 0] = sc[0]
    box_out_ref[0] = box
    int_out_ref[0] = jnp.concatenate([cv, nd_row], axis=0)


def kernel(ct_heatmaps, ct_size, ct_offset):
    B, H, W, C = ct_heatmaps.shape
    K = _K
    size_r = ct_size.reshape(B, H, 2 * W)
    off_r = ct_offset.reshape(B, H, 2 * W)

    body = functools.partial(_detgen_kernel, H=H, W=W, C=C, K=K)
    sc, box, ints = pl.pallas_call(
        body,
        grid=(B,),
        in_specs=[
            pl.BlockSpec((1, H, W, C), lambda b: (b, 0, 0, 0)),
            pl.BlockSpec((1, H, 2 * W), lambda b: (b, 0, 0)),
            pl.BlockSpec((1, H, 2 * W), lambda b: (b, 0, 0)),
        ],
        out_specs=[
            pl.BlockSpec((1, 1, H), lambda b: (b, 0, 0)),
            pl.BlockSpec((1, H, 4), lambda b: (b, 0, 0)),
            pl.BlockSpec((1, 2, H), lambda b: (b, 0, 0)),
        ],
        out_shape=[
            jax.ShapeDtypeStruct((B, 1, H), jnp.float32),
            jax.ShapeDtypeStruct((B, H, 4), jnp.float32),
            jax.ShapeDtypeStruct((B, 2, H), jnp.int32),
        ],
        scratch_shapes=[
            pltpu.VMEM((H * W // 8, 8, C), jnp.float32),
            pltpu.VMEM(((K + 7) // 8, 8, C), jnp.float32),
            pltpu.VMEM(((K + 7) // 8, 8, C), jnp.int32),
            pltpu.SMEM((K,), jnp.int32),
        ],
        compiler_params=pltpu.CompilerParams(
            dimension_semantics=("parallel",),
        ),
    )(ct_heatmaps, size_r, off_r)

    boxes = box[:, :K, :]
    channel_indices = ints[:, 0, :K]
    detection_scores = sc[:, 0, :K]
    num_detections = ints[:, 1, 0]
    return boxes, channel_indices, detection_scores, num_detections


# speedup vs baseline: 2.0670x; 2.0284x over previous
import functools

import jax
import jax.numpy as jnp
from jax import lax
from jax.experimental import pallas as pl
from jax.experimental.pallas import tpu as pltpu

_K = 100
_PEAK_EPSILON = 1e-06
_BIG = 1 << 30


def _better(v, f, pv, pf):
    return (v > pv) | ((v == pv) & (f < pf))


def _lane_stage(v, f, j, sel_mask, flip):
    pv_lo = jnp.concatenate([v[:, j:], v[:, :j]], axis=1)
    pv_hi = jnp.concatenate([v[:, -j:], v[:, :-j]], axis=1)
    pf_lo = jnp.concatenate([f[:, j:], f[:, :j]], axis=1)
    pf_hi = jnp.concatenate([f[:, -j:], f[:, :-j]], axis=1)
    lane = lax.broadcasted_iota(jnp.int32, (1, v.shape[1]), 1)
    lowbit = (lane & j) == 0
    pv = jnp.where(lowbit, pv_lo, pv_hi)
    pf = jnp.where(lowbit, pf_lo, pf_hi)
    bw = _better(v, f, pv, pf)
    keep = (sel_mask ^ flip) == bw
    return jnp.where(keep, v, pv), jnp.where(keep, f, pf)


def _row_sort(v, f, flip):
    L = v.shape[1]
    lane = lax.broadcasted_iota(jnp.int32, (1, L), 1)
    k = 2
    while k <= L:
        j = k // 2
        while j >= 1:
            if k == L:
                sel = (lane & j) == 0
            else:
                sel = ((lane & k) == 0) == ((lane & j) == 0)
            v, f = _lane_stage(v, f, j, sel, flip)
            j //= 2
        k *= 2
    return v, f


def _clean(v, f, flip):
    L = v.shape[1]
    lane = lax.broadcasted_iota(jnp.int32, (1, L), 1)
    j = L // 2
    while j >= 1:
        v, f = _lane_stage(v, f, j, (lane & j) == 0, flip)
        j //= 2
    return v, f


def _half_flip(rows):
    r = lax.broadcasted_iota(jnp.int32, (rows, 1), 0)
    return r >= (rows // 2) if rows > 1 else r < 0


def _top128(v, f):
    R = v.shape[0]
    v, f = _row_sort(v, f, _half_flip(R))
    while R > 1:
        half = R // 2
        bw = _better(v[:half], f[:half], v[half:], f[half:])
        v = jnp.where(bw, v[:half], v[half:])
        f = jnp.where(bw, f[:half], f[half:])
        R = half
        v, f = _clean(v, f, _half_flip(R))
    return v, f


def _detgen_kernel(heat_ref, size_ref, off_ref,
                   sc_out_ref, box_out_ref, int_out_ref,
                   peaks_ref, cand_ref, fid_ref, pixrow_ref, pix_smem,
                   dma_sem, *, H, W, C, K):
    HW = H * W
    x = heat_ref[0]
    p = jax.nn.sigmoid(x)

    neg_w = jnp.full((H, 1, C), -jnp.inf, dtype=jnp.float32)
    left = jnp.concatenate([neg_w, p[:, :-1, :]], axis=1)
    right = jnp.concatenate([p[:, 1:, :], neg_w], axis=1)
    mw = jnp.maximum(p, jnp.maximum(left, right))
    neg_h = jnp.full((1, W, C), -jnp.inf, dtype=jnp.float32)
    up = jnp.concatenate([neg_h, mw[:-1]], axis=0)
    dn = jnp.concatenate([mw[1:], neg_h], axis=0)
    m = jnp.maximum(mw, jnp.maximum(up, dn))

    peaks = jnp.where(jnp.abs(p - m) < _PEAK_EPSILON, p, 0.0)
    peaks_ref[...] = peaks.reshape(HW // 8, 8, C)

    colmax = jnp.max(peaks, axis=2)

    cm = colmax.reshape(HW // 128, 128)
    pix_iota = (lax.broadcasted_iota(jnp.int32, cm.shape, 0) * 128
                + lax.broadcasted_iota(jnp.int32, cm.shape, 1))
    _, pix_sorted = _top128(cm, pix_iota)
    pixrow_ref[...] = pix_sorted
    cp = pltpu.make_async_copy(pixrow_ref, pix_smem, dma_sem)
    cp.start()
    cp.wait()

    NP = cand_ref.shape[0]
    s_iota = lax.broadcasted_iota(jnp.int32, (1, 8, 128), 1)
    c_iota1 = lax.broadcasted_iota(jnp.int32, (1, 1, 128), 2)
    cpad = jnp.full((1, 1, 128 - C), -jnp.inf, dtype=jnp.float32)
    in_c = c_iota1 < C

    def gather_cand_body(j, _):
        base = j * 8
        rows, fids = [], []
        for t in range(8):
            pixj = pix_smem[0, jnp.minimum(base + t, K - 1)]
            g = pixj // 8
            s = pixj - g * 8
            page = peaks_ref[pl.ds(g, 1)]
            row = jnp.max(jnp.where(s_iota[:, :, :C] == s, page, -jnp.inf),
                          axis=1, keepdims=True)
            rows.append(jnp.concatenate([row, cpad], axis=2))
            fids.append(jnp.where(in_c, pixj * C + c_iota1, _BIG))
        valid = s_iota < K - base
        cand_ref[pl.ds(j, 1)] = jnp.where(
            valid, jnp.concatenate(rows, axis=1), -1.0)
        fid_ref[pl.ds(j, 1)] = jnp.where(
            valid, jnp.concatenate(fids, axis=1), _BIG)
        return 0

    lax.fori_loop(0, NP, gather_cand_body, 0)

    cand = cand_ref[...].reshape(NP * 8, 128)
    cfid = fid_ref[...].reshape(NP * 8, 128)
    sc, fid = _top128(cand, cfid)

    q = fid // C
    yv = q // W
    xv = q - yv * W
    cv = fid - q * C

    qT = q.reshape(128, 1)
    yT = qT // W
    xT = qT - yT * W
    lane_h = lax.broadcasted_iota(jnp.int32, (128, H), 1)
    onehot = (yT == lane_h).astype(jnp.float32)
    size_rows = jnp.dot(onehot, size_ref[0],
                        preferred_element_type=jnp.float32)
    off_rows = jnp.dot(onehot, off_ref[0],
                       preferred_element_type=jnp.float32)
    lane2 = lax.broadcasted_iota(jnp.int32, (128, 2 * W), 1)
    sel_h = lane2 == 2 * xT
    sel_w = lane2 == 2 * xT + 1
    zf = jnp.float32(0)
    h = jnp.sum(jnp.where(sel_h, size_rows, zf), axis=1, keepdims=True)
    w = jnp.sum(jnp.where(sel_w, size_rows, zf), axis=1, keepdims=True)
    yo = jnp.sum(jnp.where(sel_h, off_rows, zf), axis=1, keepdims=True)
    xo = jnp.sum(jnp.where(sel_w, off_rows, zf), axis=1, keepdims=True)

    yf = yT.astype(jnp.float32)
    xf = xT.astype(jnp.float32)
    hh = jnp.maximum(h, 0.0)
    ww = jnp.maximum(w, 0.0)
    Hf = jnp.float32(H)
    Wf = jnp.float32(W)
    ymin = jnp.clip(yf + yo - hh / 2.0, 0.0, Hf)
    xmin = jnp.clip(xf + xo - ww / 2.0, 0.0, Wf)
    ymax = jnp.clip(yf + yo + hh / 2.0, 0.0, Hf)
    xmax = jnp.clip(xf + xo + ww / 2.0, 0.0, Wf)
    box = jnp.concatenate([ymin, xmin, ymax, xmax], axis=1)
    box = jnp.clip(box * 4.0 / 512.0, 0.0, 1.0)

    k_iota = lax.broadcasted_iota(jnp.int32, (1, 128), 1)
    nd = jnp.sum(jnp.where((sc > 0.0) & (k_iota < K), 1, 0))
    nd_row = jnp.where(k_iota == 0, nd, 0)

    sc_out_ref[0, 0] = sc[0]
    box_out_ref[0] = box
    int_out_ref[0] = jnp.concatenate([cv, nd_row], axis=0)


def kernel(ct_heatmaps, ct_size, ct_offset):
    B, H, W, C = ct_heatmaps.shape
    K = _K
    size_r = ct_size.reshape(B, H, 2 * W)
    off_r = ct_offset.reshape(B, H, 2 * W)

    body = functools.partial(_detgen_kernel, H=H, W=W, C=C, K=K)
    sc, box, ints = pl.pallas_call(
        body,
        grid=(B,),
        in_specs=[
            pl.BlockSpec((1, H, W, C), lambda b: (b, 0, 0, 0)),
            pl.BlockSpec((1, H, 2 * W), lambda b: (b, 0, 0)),
            pl.BlockSpec((1, H, 2 * W), lambda b: (b, 0, 0)),
        ],
        out_specs=[
            pl.BlockSpec((1, 1, 128), lambda b: (b, 0, 0)),
            pl.BlockSpec((1, 128, 4), lambda b: (b, 0, 0)),
            pl.BlockSpec((1, 2, 128), lambda b: (b, 0, 0)),
        ],
        out_shape=[
            jax.ShapeDtypeStruct((B, 1, 128), jnp.float32),
            jax.ShapeDtypeStruct((B, 128, 4), jnp.float32),
            jax.ShapeDtypeStruct((B, 2, 128), jnp.int32),
        ],
        scratch_shapes=[
            pltpu.VMEM((H * W // 8, 8, C), jnp.float32),
            pltpu.VMEM((16, 8, 128), jnp.float32),
            pltpu.VMEM((16, 8, 128), jnp.int32),
            pltpu.VMEM((1, 128), jnp.int32),
            pltpu.SMEM((1, 128), jnp.int32),
            pltpu.SemaphoreType.DMA,
        ],
        compiler_params=pltpu.CompilerParams(
            dimension_semantics=("parallel",),
        ),
    )(ct_heatmaps, size_r, off_r)

    boxes = box[:, :K, :]
    channel_indices = ints[:, 0, :K]
    detection_scores = sc[:, 0, :K]
    num_detections = ints[:, 1, 0]
    return boxes, channel_indices, detection_scores, num_detections


# speedup vs baseline: 2.4068x; 1.1644x over previous
import functools

import jax
import jax.numpy as jnp
from jax import lax
from jax.experimental import pallas as pl
from jax.experimental.pallas import tpu as pltpu

_K = 100
_PEAK_EPSILON = 1e-06
_BIG = 1 << 30


def _better(v, f, pv, pf):
    return (v > pv) | ((v == pv) & (f < pf))


def _row_stage(v, f, j, sel_mask, flip):
    pv_lo = jnp.concatenate([v[j:], v[:j]], axis=0)
    pv_hi = jnp.concatenate([v[-j:], v[:-j]], axis=0)
    pf_lo = jnp.concatenate([f[j:], f[:j]], axis=0)
    pf_hi = jnp.concatenate([f[-j:], f[:-j]], axis=0)
    row = lax.broadcasted_iota(jnp.int32, (v.shape[0], 1), 0)
    lowbit = (row & j) == 0
    pv = jnp.where(lowbit, pv_lo, pv_hi)
    pf = jnp.where(lowbit, pf_lo, pf_hi)
    bw = _better(v, f, pv, pf)
    keep = (sel_mask ^ flip) == bw
    return jnp.where(keep, v, pv), jnp.where(keep, f, pf)


def _col_sort(v, f, flip):
    R = v.shape[0]
    row = lax.broadcasted_iota(jnp.int32, (R, 1), 0)
    k = 2
    while k <= R:
        j = k // 2
        while j >= 1:
            if k == R:
                sel = (row & j) == 0
            else:
                sel = ((row & k) == 0) == ((row & j) == 0)
            v, f = _row_stage(v, f, j, sel, flip)
            j //= 2
        k *= 2
    return v, f


def _col_clean(v, f, flip):
    R = v.shape[0]
    row = lax.broadcasted_iota(jnp.int32, (R, 1), 0)
    j = R // 2
    while j >= 1:
        v, f = _row_stage(v, f, j, (row & j) == 0, flip)
        j //= 2
    return v, f


def _lane_fold(v, f, stride):
    pv_lo = jnp.concatenate([v[:, stride:], v[:, :stride]], axis=1)
    pv_hi = jnp.concatenate([v[:, -stride:], v[:, :-stride]], axis=1)
    pf_lo = jnp.concatenate([f[:, stride:], f[:, :stride]], axis=1)
    pf_hi = jnp.concatenate([f[:, -stride:], f[:, :-stride]], axis=1)
    lane = lax.broadcasted_iota(jnp.int32, (1, v.shape[1]), 1)
    lowbit = (lane & stride) == 0
    pv = jnp.where(lowbit, pv_lo, pv_hi)
    pf = jnp.where(lowbit, pf_lo, pf_hi)
    bw = _better(v, f, pv, pf)
    keep = lowbit == bw
    return jnp.where(keep, v, pv), jnp.where(keep, f, pf)


def _topr_col(v, f):
    L = v.shape[1]
    lane = lax.broadcasted_iota(jnp.int32, (1, L), 1)
    stride = L // 2
    v, f = _col_sort(v, f, (lane & stride) != 0 if stride else lane < 0)
    while stride >= 1:
        v, f = _lane_fold(v, f, stride)
        stride //= 2
        flip = (lane & stride) != 0 if stride else lane < 0
        v, f = _col_clean(v, f, flip)
    return v[:, :1], f[:, :1]


def _detgen_kernel(heat_ref, size_ref, off_ref,
                   sc_out_ref, box_out_ref, int_out_ref,
                   peaks_ref, cand_ref, fid_ref, pixcol_ref, pix_smem,
                   dma_sem, *, H, W, C, K):
    x = heat_ref[0]
    p = jax.nn.sigmoid(x)

    neg_w = jnp.full((H, 1, C), -jnp.inf, dtype=jnp.float32)
    left = jnp.concatenate([neg_w, p[:, :-1, :]], axis=1)
    right = jnp.concatenate([p[:, 1:, :], neg_w], axis=1)
    mw = jnp.maximum(p, jnp.maximum(left, right))
    neg_h = jnp.full((1, W, C), -jnp.inf, dtype=jnp.float32)
    up = jnp.concatenate([neg_h, mw[:-1]], axis=0)
    dn = jnp.concatenate([mw[1:], neg_h], axis=0)
    m = jnp.maximum(mw, jnp.maximum(up, dn))

    peaks = jnp.where(jnp.abs(p - m) < _PEAK_EPSILON, p, 0.0)
    peaks_ref[...] = peaks

    colmax = jnp.max(peaks, axis=2)

    pix_iota = (lax.broadcasted_iota(jnp.int32, (H, W), 0) * W
                + lax.broadcasted_iota(jnp.int32, (H, W), 1))
    _, pix_sorted = _topr_col(colmax, pix_iota)
    pixcol_ref[...] = pix_sorted
    cp = pltpu.make_async_copy(pixcol_ref, pix_smem, dma_sem)
    cp.start()
    cp.wait()

    NP = cand_ref.shape[0]
    s_iota = lax.broadcasted_iota(jnp.int32, (1, 8, 128), 1)
    c_iota1 = lax.broadcasted_iota(jnp.int32, (1, 1, 128), 2)
    cpad = jnp.full((1, 1, 128 - C), -jnp.inf, dtype=jnp.float32)
    in_c = c_iota1 < C

    def gather_cand_body(j, _):
        base = j * 8
        rows, fids = [], []
        for t in range(8):
            pixj = pix_smem[jnp.minimum(base + t, K - 1), 0]
            y = pixj // W
            xx = pixj - y * W
            row = peaks_ref[pl.ds(y, 1), pl.ds(xx, 1), :]
            rows.append(jnp.concatenate([row, cpad], axis=2))
            fids.append(jnp.where(in_c, pixj * C + c_iota1, _BIG))
        valid = s_iota < K - base
        cand_ref[pl.ds(j, 1)] = jnp.where(
            valid, jnp.concatenate(rows, axis=1), -1.0)
        fid_ref[pl.ds(j, 1)] = jnp.where(
            valid, jnp.concatenate(fids, axis=1), _BIG)
        return 0

    lax.fori_loop(0, NP, gather_cand_body, 0)

    cand = cand_ref[...].reshape(NP * 8, 128)
    cfid = fid_ref[...].reshape(NP * 8, 128)
    sc, fid = _topr_col(cand, cfid)

    q = fid // C
    yT = q // W
    xT = q - yT * W
    cv = fid - q * C

    lane_h = lax.broadcasted_iota(jnp.int32, (128, H), 1)
    onehot = (yT == lane_h).astype(jnp.float32)
    size_rows = jnp.dot(onehot, size_ref[0],
                        preferred_element_type=jnp.float32)
    off_rows = jnp.dot(onehot, off_ref[0],
                       preferred_element_type=jnp.float32)
    lane2 = lax.broadcasted_iota(jnp.int32, (128, 2 * W), 1)
    sel_h = lane2 == 2 * xT
    sel_w = lane2 == 2 * xT + 1
    zf = jnp.float32(0)
    h = jnp.sum(jnp.where(sel_h, size_rows, zf), axis=1, keepdims=True)
    w = jnp.sum(jnp.where(sel_w, size_rows, zf), axis=1, keepdims=True)
    yo = jnp.sum(jnp.where(sel_h, off_rows, zf), axis=1, keepdims=True)
    xo = jnp.sum(jnp.where(sel_w, off_rows, zf), axis=1, keepdims=True)

    yf = yT.astype(jnp.float32)
    xf = xT.astype(jnp.float32)
    hh = jnp.maximum(h, 0.0)
    ww = jnp.maximum(w, 0.0)
    Hf = jnp.float32(H)
    Wf = jnp.float32(W)
    ymin = jnp.clip(yf + yo - hh / 2.0, 0.0, Hf)
    xmin = jnp.clip(xf + xo - ww / 2.0, 0.0, Wf)
    ymax = jnp.clip(yf + yo + hh / 2.0, 0.0, Hf)
    xmax = jnp.clip(xf + xo + ww / 2.0, 0.0, Wf)
    box = jnp.concatenate([ymin, xmin, ymax, xmax], axis=1)
    box = jnp.clip(box * 4.0 / 512.0, 0.0, 1.0)

    slot = lax.broadcasted_iota(jnp.int32, (128, 1), 0)
    nd = jnp.sum(jnp.where((sc > 0.0) & (slot < K), 1, 0))
    nd_col = jnp.where(slot == 0, nd, 0)

    sc_out_ref[0] = sc
    box_out_ref[0] = box
    int_out_ref[0] = jnp.concatenate([cv, nd_col], axis=1)


def kernel(ct_heatmaps, ct_size, ct_offset):
    B, H, W, C = ct_heatmaps.shape
    K = _K
    size_r = ct_size.reshape(B, H, 2 * W)
    off_r = ct_offset.reshape(B, H, 2 * W)

    body = functools.partial(_detgen_kernel, H=H, W=W, C=C, K=K)
    sc, box, ints = pl.pallas_call(
        body,
        grid=(B,),
        in_specs=[
            pl.BlockSpec((1, H, W, C), lambda b: (b, 0, 0, 0)),
            pl.BlockSpec((1, H, 2 * W), lambda b: (b, 0, 0)),
            pl.BlockSpec((1, H, 2 * W), lambda b: (b, 0, 0)),
        ],
        out_specs=[
            pl.BlockSpec((1, 128, 1), lambda b: (b, 0, 0)),
            pl.BlockSpec((1, 128, 4), lambda b: (b, 0, 0)),
            pl.BlockSpec((1, 128, 2), lambda b: (b, 0, 0)),
        ],
        out_shape=[
            jax.ShapeDtypeStruct((B, 128, 1), jnp.float32),
            jax.ShapeDtypeStruct((B, 128, 4), jnp.float32),
            jax.ShapeDtypeStruct((B, 128, 2), jnp.int32),
        ],
        scratch_shapes=[
            pltpu.VMEM((H, W, C), jnp.float32),
            pltpu.VMEM((16, 8, 128), jnp.float32),
            pltpu.VMEM((16, 8, 128), jnp.int32),
            pltpu.VMEM((H, 1), jnp.int32),
            pltpu.SMEM((H, 1), jnp.int32),
            pltpu.SemaphoreType.DMA,
        ],
        compiler_params=pltpu.CompilerParams(
            dimension_semantics=("parallel",),
        ),
    )(ct_heatmaps, size_r, off_r)

    boxes = box[:, :K, :]
    channel_indices = ints[:, :K, 0]
    detection_scores = sc[:, :K, 0]
    num_detections = ints[:, 0, 1]
    return boxes, channel_indices, detection_scores, num_detections


# speedup vs baseline: 2.8072x; 1.1663x over previous
import functools

import jax
import jax.numpy as jnp
from jax import lax
from jax.experimental import pallas as pl
from jax.experimental.pallas import tpu as pltpu

_K = 100
_PEAK_EPSILON = 1e-06
_BIG = 1 << 30


def _better(v, f, pv, pf, tie):
    if tie:
        return (v > pv) | ((v == pv) & (f < pf))
    return v > pv


def _row_stage(v, f, j, sel_mask, flip, tie):
    pv_lo = jnp.concatenate([v[j:], v[:j]], axis=0)
    pv_hi = jnp.concatenate([v[-j:], v[:-j]], axis=0)
    pf_lo = jnp.concatenate([f[j:], f[:j]], axis=0)
    pf_hi = jnp.concatenate([f[-j:], f[:-j]], axis=0)
    row = lax.broadcasted_iota(jnp.int32, (v.shape[0], 1), 0)
    lowbit = (row & j) == 0
    pv = jnp.where(lowbit, pv_lo, pv_hi)
    pf = jnp.where(lowbit, pf_lo, pf_hi)
    bw = _better(v, f, pv, pf, tie)
    keep = (sel_mask ^ flip) == bw
    return jnp.where(keep, v, pv), jnp.where(keep, f, pf)


def _col_sort(v, f, flip, tie):
    R = v.shape[0]
    row = lax.broadcasted_iota(jnp.int32, (R, 1), 0)
    k = 2
    while k <= R:
        j = k // 2
        while j >= 1:
            if k == R:
                sel = (row & j) == 0
            else:
                sel = ((row & k) == 0) == ((row & j) == 0)
            v, f = _row_stage(v, f, j, sel, flip, tie)
            j //= 2
        k *= 2
    return v, f


def _col_clean(v, f, flip, tie):
    R = v.shape[0]
    row = lax.broadcasted_iota(jnp.int32, (R, 1), 0)
    j = R // 2
    while j >= 1:
        v, f = _row_stage(v, f, j, (row & j) == 0, flip, tie)
        j //= 2
    return v, f


def _lane_fold(v, f, stride, tie):
    pv_lo = jnp.concatenate([v[:, stride:], v[:, :stride]], axis=1)
    pv_hi = jnp.concatenate([v[:, -stride:], v[:, :-stride]], axis=1)
    pf_lo = jnp.concatenate([f[:, stride:], f[:, :stride]], axis=1)
    pf_hi = jnp.concatenate([f[:, -stride:], f[:, :-stride]], axis=1)
    lane = lax.broadcasted_iota(jnp.int32, (1, v.shape[1]), 1)
    lowbit = (lane & stride) == 0
    pv = jnp.where(lowbit, pv_lo, pv_hi)
    pf = jnp.where(lowbit, pf_lo, pf_hi)
    bw = _better(v, f, pv, pf, tie)
    keep = lowbit == bw
    return jnp.where(keep, v, pv), jnp.where(keep, f, pf)


def _topr_col(v, f, tie=True):
    L = v.shape[1]
    lane = lax.broadcasted_iota(jnp.int32, (1, L), 1)
    stride = L // 2
    v, f = _col_sort(v, f, (lane & stride) != 0 if stride else lane < 0, tie)
    while stride >= 1:
        v, f = _lane_fold(v, f, stride, tie)
        stride //= 2
        flip = (lane & stride) != 0 if stride else lane < 0
        v, f = _col_clean(v, f, flip, tie)
    return v[:, :1], f[:, :1]


def _detgen_kernel(heat_ref, size_ref, off_ref,
                   sc_out_ref, box_out_ref, int_out_ref,
                   peaks_ref, cand_ref, fid_ref, pixcol_ref, pix_smem,
                   dma_sem, *, H, W, C, K):
    x = heat_ref[0]
    p = jax.nn.sigmoid(x)

    neg_w = jnp.full((H, 1, C), -jnp.inf, dtype=jnp.float32)
    left = jnp.concatenate([neg_w, p[:, :-1, :]], axis=1)
    right = jnp.concatenate([p[:, 1:, :], neg_w], axis=1)
    mw = jnp.maximum(p, jnp.maximum(left, right))
    neg_h = jnp.full((1, W, C), -jnp.inf, dtype=jnp.float32)
    up = jnp.concatenate([neg_h, mw[:-1]], axis=0)
    dn = jnp.concatenate([mw[1:], neg_h], axis=0)
    m = jnp.maximum(mw, jnp.maximum(up, dn))

    peaks = jnp.where(jnp.abs(p - m) < _PEAK_EPSILON, p, 0.0)
    peaks_ref[...] = peaks

    colmax = jnp.max(peaks, axis=2)

    pix_iota = (lax.broadcasted_iota(jnp.int32, (H, W), 0) * W
                + lax.broadcasted_iota(jnp.int32, (H, W), 1))
    _, pix_sorted = _topr_col(colmax, pix_iota, tie=False)
    pixcol_ref[...] = pix_sorted
    cp = pltpu.make_async_copy(pixcol_ref, pix_smem, dma_sem)
    cp.start()
    cp.wait()

    NP = cand_ref.shape[0]
    s_iota = lax.broadcasted_iota(jnp.int32, (1, 8, 128), 1)
    c_iota1 = lax.broadcasted_iota(jnp.int32, (1, 1, 128), 2)
    cpad = jnp.full((1, 1, 128 - C), -jnp.inf, dtype=jnp.float32)
    in_c = c_iota1 < C

    for j in range(NP):
        base = j * 8
        rows, fids = [], []
        for t in range(8):
            pixj = pix_smem[min(base + t, K - 1), 0]
            y = pixj // W
            xx = pixj - y * W
            row = peaks_ref[pl.ds(y, 1), pl.ds(xx, 1), :]
            rows.append(jnp.concatenate([row, cpad], axis=2))
            fids.append(jnp.where(in_c, pixj * C + c_iota1, _BIG))
        valid = s_iota < K - base
        cand_ref[j] = jnp.where(
            valid, jnp.concatenate(rows, axis=1), -1.0)[0]
        fid_ref[j] = jnp.where(
            valid, jnp.concatenate(fids, axis=1), _BIG)[0]

    cand = cand_ref[...].reshape(NP * 8, 128)
    cfid = fid_ref[...].reshape(NP * 8, 128)
    sc, fid = _topr_col(cand, cfid)

    q = fid // C
    yT = q // W
    xT = q - yT * W
    cv = fid - q * C

    lane_h = lax.broadcasted_iota(jnp.int32, (128, H), 1)
    onehot = (yT == lane_h).astype(jnp.float32)
    size_rows = jnp.dot(onehot, size_ref[0],
                        preferred_element_type=jnp.float32)
    off_rows = jnp.dot(onehot, off_ref[0],
                       preferred_element_type=jnp.float32)
    lane2 = lax.broadcasted_iota(jnp.int32, (128, 2 * W), 1)
    sel_h = lane2 == 2 * xT
    sel_w = lane2 == 2 * xT + 1
    zf = jnp.float32(0)
    h = jnp.sum(jnp.where(sel_h, size_rows, zf), axis=1, keepdims=True)
    w = jnp.sum(jnp.where(sel_w, size_rows, zf), axis=1, keepdims=True)
    yo = jnp.sum(jnp.where(sel_h, off_rows, zf), axis=1, keepdims=True)
    xo = jnp.sum(jnp.where(sel_w, off_rows, zf), axis=1, keepdims=True)

    yf = yT.astype(jnp.float32)
    xf = xT.astype(jnp.float32)
    hh = jnp.maximum(h, 0.0)
    ww = jnp.maximum(w, 0.0)
    Hf = jnp.float32(H)
    Wf = jnp.float32(W)
    ymin = jnp.clip(yf + yo - hh / 2.0, 0.0, Hf)
    xmin = jnp.clip(xf + xo - ww / 2.0, 0.0, Wf)
    ymax = jnp.clip(yf + yo + hh / 2.0, 0.0, Hf)
    xmax = jnp.clip(xf + xo + ww / 2.0, 0.0, Wf)
    box = jnp.concatenate([ymin, xmin, ymax, xmax], axis=1)
    box = jnp.clip(box * 4.0 / 512.0, 0.0, 1.0)

    slot = lax.broadcasted_iota(jnp.int32, (128, 1), 0)
    nd = jnp.sum(jnp.where((sc > 0.0) & (slot < K), 1, 0))
    nd_col = jnp.where(slot == 0, nd, 0)

    sc_out_ref[0] = sc
    box_out_ref[0] = box
    int_out_ref[0] = jnp.concatenate([cv, nd_col], axis=1)


def kernel(ct_heatmaps, ct_size, ct_offset):
    B, H, W, C = ct_heatmaps.shape
    K = _K
    size_r = ct_size.reshape(B, H, 2 * W)
    off_r = ct_offset.reshape(B, H, 2 * W)

    body = functools.partial(_detgen_kernel, H=H, W=W, C=C, K=K)
    sc, box, ints = pl.pallas_call(
        body,
        grid=(B,),
        in_specs=[
            pl.BlockSpec((1, H, W, C), lambda b: (b, 0, 0, 0)),
            pl.BlockSpec((1, H, 2 * W), lambda b: (b, 0, 0)),
            pl.BlockSpec((1, H, 2 * W), lambda b: (b, 0, 0)),
        ],
        out_specs=[
            pl.BlockSpec((1, 128, 1), lambda b: (b, 0, 0)),
            pl.BlockSpec((1, 128, 4), lambda b: (b, 0, 0)),
            pl.BlockSpec((1, 128, 2), lambda b: (b, 0, 0)),
        ],
        out_shape=[
            jax.ShapeDtypeStruct((B, 128, 1), jnp.float32),
            jax.ShapeDtypeStruct((B, 128, 4), jnp.float32),
            jax.ShapeDtypeStruct((B, 128, 2), jnp.int32),
        ],
        scratch_shapes=[
            pltpu.VMEM((H, W, C), jnp.float32),
            pltpu.VMEM((16, 8, 128), jnp.float32),
            pltpu.VMEM((16, 8, 128), jnp.int32),
            pltpu.VMEM((H, 1), jnp.int32),
            pltpu.SMEM((H, 1), jnp.int32),
            pltpu.SemaphoreType.DMA,
        ],
        compiler_params=pltpu.CompilerParams(
            dimension_semantics=("parallel",),
        ),
    )(ct_heatmaps, size_r, off_r)

    boxes = box[:, :K, :]
    channel_indices = ints[:, :K, 0]
    detection_scores = sc[:, :K, 0]
    num_detections = ints[:, 0, 1]
    return boxes, channel_indices, detection_scores, num_detections
